# baseline (speedup 1.0000x reference)
import functools

import jax
import jax.numpy as jnp
from jax import lax
from jax.experimental import pallas as pl
from jax.experimental.pallas import tpu as pltpu
from jax.experimental.pallas import tpu_sc as plsc

N = 10000
E = 320000
D = 128
L = 4
V = 1001
H = 256

NC = 2
NS = 16
NW = NC * NS

EPW = E // NW
SEG_CH, SEG_IG, SEG_NIG = 125, 8, 10
EA_CH, EA_IG, EA_NIG = 100, 10, 10

NPAD = 10240
RPT = NPAD // NS
ZCH = 80
NZCH = RPT // ZCH

NP = 10240
RPW = NP // NW
ECH = 80
NECH = RPW // ECH

_mesh = plsc.VectorSubcoreMesh(
    core_axis_name="c", subcore_axis_name="s", num_cores=NC, num_subcores=NS)

_f32 = jnp.float32


def _zero_rows_buf(buf, nrows):
  z = jnp.zeros((16,), _f32)

  @plsc.parallel_loop(0, nrows, unroll=4)
  def _(r):
    for j in range(D // 16):
      buf[r, pl.ds(j * 16, 16)] = z


def _make_seg(with_ea):
  CH, IG, NIG = (EA_CH, EA_IG, EA_NIG) if with_ea else (SEG_CH, SEG_IG,
                                                        SEG_NIG)
  scratch = [
      pltpu.VMEM((2, IG, CH), jnp.int32),
      pltpu.VMEM((2, IG, CH), jnp.int32),
      pltpu.VMEM((CH, D), _f32),
      pltpu.VMEM((CH, D), _f32),
      pltpu.VMEM_SHARED((NPAD, D), _f32),
      pltpu.SemaphoreType.DMA,
      pltpu.SemaphoreType.DMA,
      pltpu.SemaphoreType.DMA,
      pltpu.SemaphoreType.DMA,
  ]
  if with_ea:
    scratch.insert(4, pltpu.VMEM((CH, D), _f32))
    scratch.append(pltpu.SemaphoreType.DMA)

  def body(*refs):
    if with_ea:
      (h_hbm, src_hbm, dst_hbm, ea_hbm, out_hbm,
       sidx, didx, rows0, rows1, earows, aggr, gsem0, gsem1, ssem, isem,
       easem) = refs
    else:
      (h_hbm, src_hbm, dst_hbm, out_hbm,
       sidx, didx, rows0, rows1, aggr, gsem0, gsem1, ssem, isem) = refs
    rowbuf = (rows0, rows1)
    gsems = (gsem0, gsem1)
    c = lax.axis_index("c")
    s = lax.axis_index("s")
    w = s * NC + c

    _zero_rows_buf(rows0, ZCH)
    for q in range(NZCH):
      pltpu.sync_copy(rows0.at[pl.ds(0, ZCH)],
                      aggr.at[pl.ds(s * RPT + q * ZCH, ZCH)])
    plsc.subcore_barrier()

    pltpu.sync_copy(src_hbm.at[w, 0], sidx.at[0])
    pltpu.sync_copy(dst_hbm.at[w, 0], didx.at[0])
    pltpu.async_copy(h_hbm.at[sidx.at[0, 0]], rows0, gsem0)
    if with_ea:
      pltpu.async_copy(ea_hbm.at[w, 0, 0], earows, easem)

    def group(grp, _):
      gpar = lax.rem(grp, 2)
      ngpar = 1 - gpar

      for j in range(IG):
        par = j % 2
        rbuf = rowbuf[par]
        nbuf = rowbuf[1 - par]

        if j > 0:
          pltpu.make_async_copy(nbuf, aggr.at[didx.at[gpar, j]],
                                ssem).wait()
        else:
          @pl.when(grp > 0)
          def _():
            pltpu.make_async_copy(nbuf, aggr.at[didx.at[gpar, j]],
                                  ssem).wait()

          @pl.when(grp + 1 < NIG)
          def _():
            pltpu.async_copy(src_hbm.at[w, grp + 1], sidx.at[ngpar], isem)
            pltpu.async_copy(dst_hbm.at[w, grp + 1], didx.at[ngpar], isem)

        if j < IG - 1:
          pltpu.async_copy(h_hbm.at[sidx.at[gpar, j + 1]], nbuf,
                           gsems[1 - par])
        else:
          @pl.when(grp + 1 < NIG)
          def _():
            pltpu.make_async_copy(src_hbm.at[w, grp + 1], sidx.at[ngpar],
                                  isem).wait()
            pltpu.make_async_copy(dst_hbm.at[w, grp + 1], didx.at[ngpar],
                                  isem).wait()
            pltpu.async_copy(h_hbm.at[sidx.at[ngpar, 0]], nbuf,
                             gsems[1 - par])

        pltpu.make_async_copy(h_hbm.at[sidx.at[gpar, j]], rbuf,
                              gsems[par]).wait()

        if with_ea:
          pltpu.make_async_copy(ea_hbm.at[w, 0, 0], earows, easem).wait()

          @plsc.parallel_loop(0, CH, unroll=8)
          def _(r):
            for k in range(D // 16):
              sl = pl.ds(k * 16, 16)
              rbuf[r, sl] = jnp.maximum(rbuf[r, sl] + earows[r, sl], 0.0)

        pltpu.async_copy(rbuf, aggr.at[didx.at[gpar, j]], ssem, add=True)

        if with_ea:
          if j < IG - 1:
            pltpu.async_copy(ea_hbm.at[w, grp, j + 1], earows, easem)
          else:
            @pl.when(grp + 1 < NIG)
            def _():
              pltpu.async_copy(ea_hbm.at[w, grp + 1, 0], earows, easem)
      return 0

    lax.fori_loop(0, NIG, group, 0)
    pltpu.make_async_copy(rowbuf[(IG - 1) % 2], aggr.at[didx.at[0, 0]],
                          ssem).wait()
    plsc.subcore_barrier()

    pltpu.sync_copy(aggr.at[pl.ds(s * RPT, RPT)],
                    out_hbm.at[c, pl.ds(s * RPT, RPT)])

  out_type = jax.ShapeDtypeStruct((NC, NPAD, D), _f32)
  return pl.kernel(body, out_type=out_type, mesh=_mesh, scratch_types=scratch,
                   name="seg_sum_ea" if with_ea else "seg_sum")


_seg_ea = _make_seg(True)
_seg = _make_seg(False)


VT = 1002
VTO = 1008


def _embed_body(keys_hbm, vals_hbm, i0_hbm, i1_hbm, out_hbm,
                i0v, i1v, ra, rb, tbl, sem0, sem1):
  c = lax.axis_index("c")
  s = lax.axis_index("s")
  w = s * NC + c
  @pl.when(s == 0)
  def _():
    pltpu.sync_copy(keys_hbm, tbl.at[pl.ds(0, VT)])

  @pl.when(s == 1)
  def _():
    pltpu.sync_copy(vals_hbm, tbl.at[pl.ds(VTO, VT)])

  pltpu.sync_copy(i0_hbm.at[w], i0v)
  pltpu.sync_copy(i1_hbm.at[w], i1v)
  plsc.subcore_barrier()

  pltpu.async_copy(tbl.at[i0v.at[0]], ra.at[0], sem0)
  pltpu.async_copy(tbl.at[i1v.at[0]], rb.at[0], sem1)
  for g in range(NECH):
    par = g % 2
    npar = 1 - par
    if g + 1 < NECH:
      pltpu.async_copy(tbl.at[i0v.at[g + 1]], ra.at[npar], sem0)
      pltpu.async_copy(tbl.at[i1v.at[g + 1]], rb.at[npar], sem1)
    pltpu.make_async_copy(tbl.at[i0v.at[g]], ra.at[par], sem0).wait()
    pltpu.make_async_copy(tbl.at[i1v.at[g]], rb.at[par], sem1).wait()

    @plsc.parallel_loop(0, ECH, unroll=8)
    def _(r):
      for j in range(D // 16):
        ra[par, r, pl.ds(j * 16, 16)] = (
            ra[par, r, pl.ds(j * 16, 16)] + rb[par, r, pl.ds(j * 16, 16)])
    pltpu.sync_copy(ra.at[par], out_hbm.at[pl.ds(w * RPW + g * ECH, ECH)])


_embed = pl.kernel(
    _embed_body,
    out_type=jax.ShapeDtypeStruct((NP, D), _f32),
    mesh=_mesh,
    scratch_types=[
        pltpu.VMEM((NECH, ECH), jnp.int32),
        pltpu.VMEM((NECH, ECH), jnp.int32),
        pltpu.VMEM((2, ECH, D), _f32),
        pltpu.VMEM((2, ECH, D), _f32),
        pltpu.VMEM_SHARED((VTO + VT, D), _f32),
        pltpu.SemaphoreType.DMA,
        pltpu.SemaphoreType.DMA,
    ],
    name="embed")


def _make_tc_prop(update_original, with_ln):
  def body(*refs):
    if with_ln:
      (h_ref, agg_ref, w1_ref, b1_ref, w2_ref, b2_ref, ep_ref,
       g_ref, b_ref, mk_ref, lg_ref, lb_ref, out_ref) = refs
    else:
      (h_ref, agg_ref, w1_ref, b1_ref, w2_ref, b2_ref, ep_ref,
       g_ref, b_ref, mk_ref, out_ref) = refs
    h = h_ref[...]
    agg = agg_ref[0, :N, :] + agg_ref[1, :N, :]
    z = ep_ref[...] * h + agg
    u = jnp.maximum(
        jnp.dot(z, w1_ref[...], preferred_element_type=_f32) + b1_ref[...],
        0.0)
    z2 = jnp.dot(u, w2_ref[...], preferred_element_type=_f32) + b2_ref[...]
    m = mk_ref[...]
    if update_original:
      hc = m * z2 + (1.0 - m) * h
    else:
      hc = m * h + (1.0 - m) * z2
    mu = jnp.mean(hc, axis=0, keepdims=True)
    var = jnp.mean((hc - mu) * (hc - mu), axis=0, keepdims=True)
    hb = g_ref[...] * (hc - mu) * lax.rsqrt(var + 1e-5) + b_ref[...]
    hn = jnp.maximum(hb, 0.0) + h
    if with_ln:
      mu2 = jnp.mean(hn, axis=1, keepdims=True)
      v2 = jnp.mean((hn - mu2) * (hn - mu2), axis=1, keepdims=True)
      hn = lg_ref[...] * (hn - mu2) * lax.rsqrt(v2 + 1e-5) + lb_ref[...]
    out_ref[...] = hn

  return pl.pallas_call(
      body, out_shape=jax.ShapeDtypeStruct((N, D), _f32))


_tc_orig = _make_tc_prop(True, False)
_tc_exp = _make_tc_prop(False, False)
_tc_orig_ln = _make_tc_prop(True, True)


def kernel(x, edge_index, edge_attr, expander_edge_index, expander_node_mask,
           params):
  p = params
  maskf = expander_node_mask.astype(_f32)[:, None]

  keys_z = jnp.concatenate([p['keys_table'], jnp.zeros((1, D), _f32)], axis=0)
  vals_z = jnp.concatenate([p['values_table'], jnp.zeros((1, D), _f32)],
                           axis=0)
  zpad = jnp.full((NP - N,), V, jnp.int32)
  i0 = jnp.concatenate(
      [jnp.where(expander_node_mask > 0, x[:, 0], V), zpad]).reshape(
          NW, NECH, ECH)
  i1 = jnp.concatenate(
      [jnp.where(expander_node_mask > 0, x[:, 1], V) + VTO,
       zpad + VTO]).reshape(NW, NECH, ECH)
  h = _embed(keys_z, vals_z, i0, i1)[:N]

  src_c = edge_index[0].reshape(NW, EA_NIG, EA_IG, EA_CH)
  dst_c = edge_index[1].reshape(NW, EA_NIG, EA_IG, EA_CH)
  ea_r = edge_attr.reshape(NW, EA_NIG, EA_IG, EA_CH, D)
  src_x = expander_edge_index[0].reshape(NW, SEG_NIG, SEG_IG, SEG_CH)
  dst_x = expander_edge_index[1].reshape(NW, SEG_NIG, SEG_IG, SEG_CH)

  def b2d(v):
    return v.reshape(1, -1)

  one = jnp.ones((1, 1), _f32)

  for l in range(L):
    agg = _seg_ea(h, src_c, dst_c, ea_r)
    h = _tc_orig(h, agg, p['conv_W1'][l], b2d(p['conv_b1'][l]),
                 p['conv_W2'][l], b2d(p['conv_b2'][l]),
                 one + p['conv_eps'][l], b2d(p['bn_gamma'][l]),
                 b2d(p['bn_beta'][l]), maskf)
    agg = _seg(h, src_x, dst_x)
    h = _tc_exp(h, agg, p['left_W1'][l], b2d(p['left_b1'][l]),
                p['left_W2'][l], b2d(p['left_b2'][l]),
                one + p['left_eps'][l], b2d(p['left_bn_gamma'][l]),
                b2d(p['left_bn_beta'][l]), maskf)
    agg = _seg(h, dst_x, src_x)
    h = _tc_orig_ln(h, agg, p['right_W1'][l], b2d(p['right_b1'][l]),
                    p['right_W2'][l], b2d(p['right_b2'][l]),
                    one + p['right_eps'][l], b2d(p['right_bn_gamma'][l]),
                    b2d(p['right_bn_beta'][l]), maskf,
                    b2d(p['ln_gamma'][l]), b2d(p['ln_beta'][l]))
  return h


# speedup vs baseline: 1.0063x; 1.0063x over previous
import functools

import jax
import jax.numpy as jnp
from jax import lax
from jax.experimental import pallas as pl
from jax.experimental.pallas import tpu as pltpu
from jax.experimental.pallas import tpu_sc as plsc

N = 10000
E = 320000
D = 128
L = 4
V = 1001
H = 256

NC = 2
NS = 16
NW = NC * NS

EPW = E // NW
SEG_CH, SEG_IG, SEG_NIG = 125, 8, 10
EA_CH, EA_IG, EA_NIG = 100, 10, 10

NPAD = 10240
RPT = NPAD // NS
ZCH = 80
NZCH = RPT // ZCH

NP = 10240
RPW = NP // NW
ECH = 80
NECH = RPW // ECH

_mesh = plsc.VectorSubcoreMesh(
    core_axis_name="c", subcore_axis_name="s", num_cores=NC, num_subcores=NS)

_f32 = jnp.float32


def _zero_rows_buf(buf, nrows):
  z = jnp.zeros((16,), _f32)

  @plsc.parallel_loop(0, nrows, unroll=4)
  def _(r):
    for j in range(D // 16):
      buf[r, pl.ds(j * 16, 16)] = z


def _make_seg(with_ea):
  CH, IG, NIG = (EA_CH, EA_IG, EA_NIG) if with_ea else (SEG_CH, SEG_IG,
                                                        SEG_NIG)
  scratch = [
      pltpu.VMEM((2, IG, CH), jnp.int32),
      pltpu.VMEM((2, IG, CH), jnp.int32),
      pltpu.VMEM((CH, D), _f32),
      pltpu.VMEM((CH, D), _f32),
      pltpu.VMEM_SHARED((NPAD, D), _f32),
      pltpu.SemaphoreType.DMA,
      pltpu.SemaphoreType.DMA,
      pltpu.SemaphoreType.DMA,
      pltpu.SemaphoreType.DMA,
  ]
  if with_ea:
    scratch.insert(4, pltpu.VMEM((CH, D), _f32))
    scratch.append(pltpu.SemaphoreType.DMA)

  def body(*refs):
    if with_ea:
      (h_hbm, src_hbm, dst_hbm, ea_hbm, out_hbm,
       sidx, didx, rows0, rows1, earows, aggr, gsem0, gsem1, ssem, isem,
       easem) = refs
    else:
      (h_hbm, src_hbm, dst_hbm, out_hbm,
       sidx, didx, rows0, rows1, aggr, gsem0, gsem1, ssem, isem) = refs
    rowbuf = (rows0, rows1)
    gsems = (gsem0, gsem1)
    c = lax.axis_index("c")
    s = lax.axis_index("s")
    w = s * NC + c

    _zero_rows_buf(rows0, ZCH)
    for q in range(NZCH):
      pltpu.sync_copy(rows0.at[pl.ds(0, ZCH)],
                      aggr.at[pl.ds(s * RPT + q * ZCH, ZCH)])
    plsc.subcore_barrier()

    pltpu.sync_copy(src_hbm.at[w, 0], sidx.at[0])
    pltpu.sync_copy(dst_hbm.at[w, 0], didx.at[0])
    pltpu.async_copy(h_hbm.at[sidx.at[0, 0]], rows0, gsem0)
    if with_ea:
      pltpu.async_copy(ea_hbm.at[w, 0, 0], earows, easem)

    def group(grp, _):
      gpar = lax.rem(grp, 2)
      ngpar = 1 - gpar

      for j in range(IG):
        par = j % 2
        rbuf = rowbuf[par]
        nbuf = rowbuf[1 - par]

        if j > 0:
          pltpu.make_async_copy(nbuf, aggr.at[didx.at[gpar, j]],
                                ssem).wait()
        else:
          @pl.when(grp > 0)
          def _():
            pltpu.make_async_copy(nbuf, aggr.at[didx.at[gpar, j]],
                                  ssem).wait()

          @pl.when(grp + 1 < NIG)
          def _():
            pltpu.async_copy(src_hbm.at[w, grp + 1], sidx.at[ngpar], isem)
            pltpu.async_copy(dst_hbm.at[w, grp + 1], didx.at[ngpar], isem)

        if j < IG - 1:
          pltpu.async_copy(h_hbm.at[sidx.at[gpar, j + 1]], nbuf,
                           gsems[1 - par])
        else:
          @pl.when(grp + 1 < NIG)
          def _():
            pltpu.make_async_copy(src_hbm.at[w, grp + 1], sidx.at[ngpar],
                                  isem).wait()
            pltpu.make_async_copy(dst_hbm.at[w, grp + 1], didx.at[ngpar],
                                  isem).wait()
            pltpu.async_copy(h_hbm.at[sidx.at[ngpar, 0]], nbuf,
                             gsems[1 - par])

        pltpu.make_async_copy(h_hbm.at[sidx.at[gpar, j]], rbuf,
                              gsems[par]).wait()

        if with_ea:
          pltpu.make_async_copy(ea_hbm.at[w, 0, 0], earows, easem).wait()

          @plsc.parallel_loop(0, CH, unroll=4)
          def _(r):
            for k in range(D // 16):
              sl = pl.ds(k * 16, 16)
              rbuf[r, sl] = jnp.maximum(rbuf[r, sl] + earows[r, sl], 0.0)

        pltpu.async_copy(rbuf, aggr.at[didx.at[gpar, j]], ssem, add=True)

        if with_ea:
          if j < IG - 1:
            pltpu.async_copy(ea_hbm.at[w, grp, j + 1], earows, easem)
          else:
            @pl.when(grp + 1 < NIG)
            def _():
              pltpu.async_copy(ea_hbm.at[w, grp + 1, 0], earows, easem)
      return 0

    lax.fori_loop(0, NIG, group, 0)
    pltpu.make_async_copy(rowbuf[(IG - 1) % 2], aggr.at[didx.at[0, 0]],
                          ssem).wait()
    plsc.subcore_barrier()

    pltpu.sync_copy(aggr.at[pl.ds(s * RPT, RPT)],
                    out_hbm.at[c, pl.ds(s * RPT, RPT)])

  out_type = jax.ShapeDtypeStruct((NC, NPAD, D), _f32)
  return pl.kernel(body, out_type=out_type, mesh=_mesh, scratch_types=scratch,
                   name="seg_sum_ea" if with_ea else "seg_sum")


_seg_ea = _make_seg(True)
_seg = _make_seg(False)


VT = 1002
VTO = 1008


def _embed_body(keys_hbm, vals_hbm, i0_hbm, i1_hbm, out_hbm,
                i0v, i1v, ra, rb, tbl, sem0, sem1):
  c = lax.axis_index("c")
  s = lax.axis_index("s")
  w = s * NC + c
  @pl.when(s == 0)
  def _():
    pltpu.sync_copy(keys_hbm, tbl.at[pl.ds(0, VT)])

  @pl.when(s == 1)
  def _():
    pltpu.sync_copy(vals_hbm, tbl.at[pl.ds(VTO, VT)])

  pltpu.sync_copy(i0_hbm.at[w], i0v)
  pltpu.sync_copy(i1_hbm.at[w], i1v)
  plsc.subcore_barrier()

  pltpu.async_copy(tbl.at[i0v.at[0]], ra.at[0], sem0)
  pltpu.async_copy(tbl.at[i1v.at[0]], rb.at[0], sem1)
  for g in range(NECH):
    par = g % 2
    npar = 1 - par
    if g + 1 < NECH:
      pltpu.async_copy(tbl.at[i0v.at[g + 1]], ra.at[npar], sem0)
      pltpu.async_copy(tbl.at[i1v.at[g + 1]], rb.at[npar], sem1)
    pltpu.make_async_copy(tbl.at[i0v.at[g]], ra.at[par], sem0).wait()
    pltpu.make_async_copy(tbl.at[i1v.at[g]], rb.at[par], sem1).wait()

    @plsc.parallel_loop(0, ECH, unroll=8)
    def _(r):
      for j in range(D // 16):
        ra[par, r, pl.ds(j * 16, 16)] = (
            ra[par, r, pl.ds(j * 16, 16)] + rb[par, r, pl.ds(j * 16, 16)])
    pltpu.sync_copy(ra.at[par], out_hbm.at[pl.ds(w * RPW + g * ECH, ECH)])


_embed = pl.kernel(
    _embed_body,
    out_type=jax.ShapeDtypeStruct((NP, D), _f32),
    mesh=_mesh,
    scratch_types=[
        pltpu.VMEM((NECH, ECH), jnp.int32),
        pltpu.VMEM((NECH, ECH), jnp.int32),
        pltpu.VMEM((2, ECH, D), _f32),
        pltpu.VMEM((2, ECH, D), _f32),
        pltpu.VMEM_SHARED((VTO + VT, D), _f32),
        pltpu.SemaphoreType.DMA,
        pltpu.SemaphoreType.DMA,
    ],
    name="embed")


def _make_tc_prop(update_original, with_ln):
  def body(*refs):
    if with_ln:
      (h_ref, agg_ref, w1_ref, b1_ref, w2_ref, b2_ref, ep_ref,
       g_ref, b_ref, mk_ref, lg_ref, lb_ref, out_ref) = refs
    else:
      (h_ref, agg_ref, w1_ref, b1_ref, w2_ref, b2_ref, ep_ref,
       g_ref, b_ref, mk_ref, out_ref) = refs
    h = h_ref[...]
    agg = agg_ref[0, :N, :] + agg_ref[1, :N, :]
    z = ep_ref[...] * h + agg
    u = jnp.maximum(
        jnp.dot(z, w1_ref[...], preferred_element_type=_f32) + b1_ref[...],
        0.0)
    z2 = jnp.dot(u, w2_ref[...], preferred_element_type=_f32) + b2_ref[...]
    m = mk_ref[...]
    if update_original:
      hc = m * z2 + (1.0 - m) * h
    else:
      hc = m * h + (1.0 - m) * z2
    mu = jnp.mean(hc, axis=0, keepdims=True)
    var = jnp.mean((hc - mu) * (hc - mu), axis=0, keepdims=True)
    hb = g_ref[...] * (hc - mu) * lax.rsqrt(var + 1e-5) + b_ref[...]
    hn = jnp.maximum(hb, 0.0) + h
    if with_ln:
      mu2 = jnp.mean(hn, axis=1, keepdims=True)
      v2 = jnp.mean((hn - mu2) * (hn - mu2), axis=1, keepdims=True)
      hn = lg_ref[...] * (hn - mu2) * lax.rsqrt(v2 + 1e-5) + lb_ref[...]
    out_ref[...] = hn

  return pl.pallas_call(
      body, out_shape=jax.ShapeDtypeStruct((N, D), _f32))


_tc_orig = _make_tc_prop(True, False)
_tc_exp = _make_tc_prop(False, False)
_tc_orig_ln = _make_tc_prop(True, True)


def kernel(x, edge_index, edge_attr, expander_edge_index, expander_node_mask,
           params):
  p = params
  maskf = expander_node_mask.astype(_f32)[:, None]

  keys_z = jnp.concatenate([p['keys_table'], jnp.zeros((1, D), _f32)], axis=0)
  vals_z = jnp.concatenate([p['values_table'], jnp.zeros((1, D), _f32)],
                           axis=0)
  zpad = jnp.full((NP - N,), V, jnp.int32)
  i0 = jnp.concatenate(
      [jnp.where(expander_node_mask > 0, x[:, 0], V), zpad]).reshape(
          NW, NECH, ECH)
  i1 = jnp.concatenate(
      [jnp.where(expander_node_mask > 0, x[:, 1], V) + VTO,
       zpad + VTO]).reshape(NW, NECH, ECH)
  h = _embed(keys_z, vals_z, i0, i1)[:N]

  src_c = edge_index[0].reshape(NW, EA_NIG, EA_IG, EA_CH)
  dst_c = edge_index[1].reshape(NW, EA_NIG, EA_IG, EA_CH)
  ea_r = edge_attr.reshape(NW, EA_NIG, EA_IG, EA_CH, D)
  src_x = expander_edge_index[0].reshape(NW, SEG_NIG, SEG_IG, SEG_CH)
  dst_x = expander_edge_index[1].reshape(NW, SEG_NIG, SEG_IG, SEG_CH)

  def b2d(v):
    return v.reshape(1, -1)

  one = jnp.ones((1, 1), _f32)

  for l in range(L):
    agg = _seg_ea(h, src_c, dst_c, ea_r)
    h = _tc_orig(h, agg, p['conv_W1'][l], b2d(p['conv_b1'][l]),
                 p['conv_W2'][l], b2d(p['conv_b2'][l]),
                 one + p['conv_eps'][l], b2d(p['bn_gamma'][l]),
                 b2d(p['bn_beta'][l]), maskf)
    agg = _seg(h, src_x, dst_x)
    h = _tc_exp(h, agg, p['left_W1'][l], b2d(p['left_b1'][l]),
                p['left_W2'][l], b2d(p['left_b2'][l]),
                one + p['left_eps'][l], b2d(p['left_bn_gamma'][l]),
                b2d(p['left_bn_beta'][l]), maskf)
    agg = _seg(h, dst_x, src_x)
    h = _tc_orig_ln(h, agg, p['right_W1'][l], b2d(p['right_b1'][l]),
                    p['right_W2'][l], b2d(p['right_b2'][l]),
                    one + p['right_eps'][l], b2d(p['right_bn_gamma'][l]),
                    b2d(p['right_bn_beta'][l]), maskf,
                    b2d(p['ln_gamma'][l]), b2d(p['ln_beta'][l]))
  return h


# speedup vs baseline: 1.0175x; 1.0111x over previous
import functools

import jax
import jax.numpy as jnp
from jax import lax
from jax.experimental import pallas as pl
from jax.experimental.pallas import tpu as pltpu
from jax.experimental.pallas import tpu_sc as plsc

N = 10000
E = 320000
D = 128
L = 4
V = 1001
H = 256

NC = 2
NS = 16
NW = NC * NS

EPW = E // NW
SEG_CH, SEG_IG, SEG_NIG = 125, 8, 10
EA_CH, EA_IG, EA_NIG = 100, 10, 10

NPAD = 10240
RPT = NPAD // NS
ZCH = 80
NZCH = RPT // ZCH

NP = 10240
RPW = NP // NW
ECH = 80
NECH = RPW // ECH

_mesh = plsc.VectorSubcoreMesh(
    core_axis_name="c", subcore_axis_name="s", num_cores=NC, num_subcores=NS)

_f32 = jnp.float32


def _zero_rows_buf(buf, nrows):
  z = jnp.zeros((16,), _f32)

  @plsc.parallel_loop(0, nrows, unroll=4)
  def _(r):
    for j in range(D // 16):
      buf[r, pl.ds(j * 16, 16)] = z


def _make_seg(with_ea):
  CH, IG, NIG = (EA_CH, EA_IG, EA_NIG) if with_ea else (SEG_CH, SEG_IG,
                                                        SEG_NIG)
  scratch = [
      pltpu.VMEM((2, IG, CH), jnp.int32),
      pltpu.VMEM((2, IG, CH), jnp.int32),
      pltpu.VMEM((CH, D), _f32),
      pltpu.VMEM((CH, D), _f32),
      pltpu.VMEM_SHARED((NPAD, D), _f32),
      pltpu.SemaphoreType.DMA,
      pltpu.SemaphoreType.DMA,
      pltpu.SemaphoreType.DMA,
      pltpu.SemaphoreType.DMA,
  ]
  if with_ea:
    scratch.insert(4, pltpu.VMEM((CH, D), _f32))
    scratch.append(pltpu.SemaphoreType.DMA)

  def body(*refs):
    if with_ea:
      (h_hbm, src_hbm, dst_hbm, ea_hbm, out_hbm,
       sidx, didx, rows0, rows1, earows, aggr, gsem0, gsem1, ssem, isem,
       easem) = refs
    else:
      (h_hbm, src_hbm, dst_hbm, out_hbm,
       sidx, didx, rows0, rows1, aggr, gsem0, gsem1, ssem, isem) = refs
    rowbuf = (rows0, rows1)
    gsems = (gsem0, gsem1)
    c = lax.axis_index("c")
    s = lax.axis_index("s")
    w = s * NC + c

    pltpu.sync_copy(src_hbm.at[w, 0], sidx.at[0])
    pltpu.sync_copy(dst_hbm.at[w, 0], didx.at[0])
    pltpu.async_copy(h_hbm.at[sidx.at[0, 0]], rows0, gsem0)
    if with_ea:
      pltpu.async_copy(ea_hbm.at[w, 0, 0], earows, easem)

    _zero_rows_buf(rows1, ZCH)
    for q in range(NZCH):
      pltpu.sync_copy(rows1.at[pl.ds(0, ZCH)],
                      aggr.at[pl.ds(s * RPT + q * ZCH, ZCH)])
    plsc.subcore_barrier()

    def group(grp, _):
      gpar = lax.rem(grp, 2)
      ngpar = 1 - gpar

      for j in range(IG):
        par = j % 2
        rbuf = rowbuf[par]
        nbuf = rowbuf[1 - par]

        if j > 0:
          pltpu.make_async_copy(nbuf, aggr.at[didx.at[gpar, j]],
                                ssem).wait()
        else:
          @pl.when(grp > 0)
          def _():
            pltpu.make_async_copy(nbuf, aggr.at[didx.at[gpar, j]],
                                  ssem).wait()

          @pl.when(grp + 1 < NIG)
          def _():
            pltpu.async_copy(src_hbm.at[w, grp + 1], sidx.at[ngpar], isem)
            pltpu.async_copy(dst_hbm.at[w, grp + 1], didx.at[ngpar], isem)

        if j < IG - 1:
          pltpu.async_copy(h_hbm.at[sidx.at[gpar, j + 1]], nbuf,
                           gsems[1 - par])
        else:
          @pl.when(grp + 1 < NIG)
          def _():
            pltpu.make_async_copy(src_hbm.at[w, grp + 1], sidx.at[ngpar],
                                  isem).wait()
            pltpu.make_async_copy(dst_hbm.at[w, grp + 1], didx.at[ngpar],
                                  isem).wait()
            pltpu.async_copy(h_hbm.at[sidx.at[ngpar, 0]], nbuf,
                             gsems[1 - par])

        pltpu.make_async_copy(h_hbm.at[sidx.at[gpar, j]], rbuf,
                              gsems[par]).wait()

        if with_ea:
          pltpu.make_async_copy(ea_hbm.at[w, 0, 0], earows, easem).wait()

          @plsc.parallel_loop(0, CH, unroll=4)
          def _(r):
            for k in range(D // 16):
              sl = pl.ds(k * 16, 16)
              rbuf[r, sl] = jnp.maximum(rbuf[r, sl] + earows[r, sl], 0.0)

        pltpu.async_copy(rbuf, aggr.at[didx.at[gpar, j]], ssem, add=True)

        if with_ea:
          if j < IG - 1:
            pltpu.async_copy(ea_hbm.at[w, grp, j + 1], earows, easem)
          else:
            @pl.when(grp + 1 < NIG)
            def _():
              pltpu.async_copy(ea_hbm.at[w, grp + 1, 0], earows, easem)
      return 0

    lax.fori_loop(0, NIG, group, 0)
    pltpu.make_async_copy(rowbuf[(IG - 1) % 2], aggr.at[didx.at[0, 0]],
                          ssem).wait()
    plsc.subcore_barrier()

    pltpu.sync_copy(aggr.at[pl.ds(s * RPT, RPT)],
                    out_hbm.at[c, pl.ds(s * RPT, RPT)])

  out_type = jax.ShapeDtypeStruct((NC, NPAD, D), _f32)
  return pl.kernel(body, out_type=out_type, mesh=_mesh, scratch_types=scratch,
                   name="seg_sum_ea" if with_ea else "seg_sum")


_seg_ea = _make_seg(True)
_seg = _make_seg(False)


VT = 1002
VTO = 1008


def _embed_body(keys_hbm, vals_hbm, i0_hbm, i1_hbm, out_hbm,
                i0v, i1v, ra, rb, tbl, sem0, sem1):
  c = lax.axis_index("c")
  s = lax.axis_index("s")
  w = s * NC + c
  @pl.when(s == 0)
  def _():
    pltpu.sync_copy(keys_hbm, tbl.at[pl.ds(0, VT)])

  @pl.when(s == 1)
  def _():
    pltpu.sync_copy(vals_hbm, tbl.at[pl.ds(VTO, VT)])

  pltpu.sync_copy(i0_hbm.at[w], i0v)
  pltpu.sync_copy(i1_hbm.at[w], i1v)
  plsc.subcore_barrier()

  pltpu.async_copy(tbl.at[i0v.at[0]], ra.at[0], sem0)
  pltpu.async_copy(tbl.at[i1v.at[0]], rb.at[0], sem1)
  for g in range(NECH):
    par = g % 2
    npar = 1 - par
    if g + 1 < NECH:
      pltpu.async_copy(tbl.at[i0v.at[g + 1]], ra.at[npar], sem0)
      pltpu.async_copy(tbl.at[i1v.at[g + 1]], rb.at[npar], sem1)
    pltpu.make_async_copy(tbl.at[i0v.at[g]], ra.at[par], sem0).wait()
    pltpu.make_async_copy(tbl.at[i1v.at[g]], rb.at[par], sem1).wait()

    @plsc.parallel_loop(0, ECH, unroll=8)
    def _(r):
      for j in range(D // 16):
        ra[par, r, pl.ds(j * 16, 16)] = (
            ra[par, r, pl.ds(j * 16, 16)] + rb[par, r, pl.ds(j * 16, 16)])
    pltpu.sync_copy(ra.at[par], out_hbm.at[pl.ds(w * RPW + g * ECH, ECH)])


_embed = pl.kernel(
    _embed_body,
    out_type=jax.ShapeDtypeStruct((NP, D), _f32),
    mesh=_mesh,
    scratch_types=[
        pltpu.VMEM((NECH, ECH), jnp.int32),
        pltpu.VMEM((NECH, ECH), jnp.int32),
        pltpu.VMEM((2, ECH, D), _f32),
        pltpu.VMEM((2, ECH, D), _f32),
        pltpu.VMEM_SHARED((VTO + VT, D), _f32),
        pltpu.SemaphoreType.DMA,
        pltpu.SemaphoreType.DMA,
    ],
    name="embed")


def _make_tc_prop(update_original, with_ln):
  def body(*refs):
    if with_ln:
      (h_ref, agg_ref, w1_ref, b1_ref, w2_ref, b2_ref, ep_ref,
       g_ref, b_ref, mk_ref, lg_ref, lb_ref, out_ref) = refs
    else:
      (h_ref, agg_ref, w1_ref, b1_ref, w2_ref, b2_ref, ep_ref,
       g_ref, b_ref, mk_ref, out_ref) = refs
    h = h_ref[...]
    agg = agg_ref[0, :N, :] + agg_ref[1, :N, :]
    z = ep_ref[...] * h + agg
    u = jnp.maximum(
        jnp.dot(z, w1_ref[...], preferred_element_type=_f32) + b1_ref[...],
        0.0)
    z2 = jnp.dot(u, w2_ref[...], preferred_element_type=_f32) + b2_ref[...]
    m = mk_ref[...]
    if update_original:
      hc = m * z2 + (1.0 - m) * h
    else:
      hc = m * h + (1.0 - m) * z2
    mu = jnp.mean(hc, axis=0, keepdims=True)
    var = jnp.mean((hc - mu) * (hc - mu), axis=0, keepdims=True)
    hb = g_ref[...] * (hc - mu) * lax.rsqrt(var + 1e-5) + b_ref[...]
    hn = jnp.maximum(hb, 0.0) + h
    if with_ln:
      mu2 = jnp.mean(hn, axis=1, keepdims=True)
      v2 = jnp.mean((hn - mu2) * (hn - mu2), axis=1, keepdims=True)
      hn = lg_ref[...] * (hn - mu2) * lax.rsqrt(v2 + 1e-5) + lb_ref[...]
    out_ref[...] = hn

  return pl.pallas_call(
      body, out_shape=jax.ShapeDtypeStruct((N, D), _f32))


_tc_orig = _make_tc_prop(True, False)
_tc_exp = _make_tc_prop(False, False)
_tc_orig_ln = _make_tc_prop(True, True)


def kernel(x, edge_index, edge_attr, expander_edge_index, expander_node_mask,
           params):
  p = params
  maskf = expander_node_mask.astype(_f32)[:, None]

  keys_z = jnp.concatenate([p['keys_table'], jnp.zeros((1, D), _f32)], axis=0)
  vals_z = jnp.concatenate([p['values_table'], jnp.zeros((1, D), _f32)],
                           axis=0)
  zpad = jnp.full((NP - N,), V, jnp.int32)
  i0 = jnp.concatenate(
      [jnp.where(expander_node_mask > 0, x[:, 0], V), zpad]).reshape(
          NW, NECH, ECH)
  i1 = jnp.concatenate(
      [jnp.where(expander_node_mask > 0, x[:, 1], V) + VTO,
       zpad + VTO]).reshape(NW, NECH, ECH)
  h = _embed(keys_z, vals_z, i0, i1)[:N]

  src_c = edge_index[0].reshape(NW, EA_NIG, EA_IG, EA_CH)
  dst_c = edge_index[1].reshape(NW, EA_NIG, EA_IG, EA_CH)
  ea_r = edge_attr.reshape(NW, EA_NIG, EA_IG, EA_CH, D)
  src_x = expander_edge_index[0].reshape(NW, SEG_NIG, SEG_IG, SEG_CH)
  dst_x = expander_edge_index[1].reshape(NW, SEG_NIG, SEG_IG, SEG_CH)

  def b2d(v):
    return v.reshape(1, -1)

  one = jnp.ones((1, 1), _f32)

  for l in range(L):
    agg = _seg_ea(h, src_c, dst_c, ea_r)
    h = _tc_orig(h, agg, p['conv_W1'][l], b2d(p['conv_b1'][l]),
                 p['conv_W2'][l], b2d(p['conv_b2'][l]),
                 one + p['conv_eps'][l], b2d(p['bn_gamma'][l]),
                 b2d(p['bn_beta'][l]), maskf)
    agg = _seg(h, src_x, dst_x)
    h = _tc_exp(h, agg, p['left_W1'][l], b2d(p['left_b1'][l]),
                p['left_W2'][l], b2d(p['left_b2'][l]),
                one + p['left_eps'][l], b2d(p['left_bn_gamma'][l]),
                b2d(p['left_bn_beta'][l]), maskf)
    agg = _seg(h, dst_x, src_x)
    h = _tc_orig_ln(h, agg, p['right_W1'][l], b2d(p['right_b1'][l]),
                    p['right_W2'][l], b2d(p['right_b2'][l]),
                    one + p['right_eps'][l], b2d(p['right_bn_gamma'][l]),
                    b2d(p['right_bn_beta'][l]), maskf,
                    b2d(p['ln_gamma'][l]), b2d(p['ln_beta'][l]))
  return h
